# Initial kernel scaffold; baseline (speedup 1.0000x reference)
import jax
import jax.numpy as jnp
from jax.experimental import pallas as pl


def kernel(x):
    raise NotImplementedError("write your pallas kernel here")


# speedup vs baseline: 2.0672x; 2.0672x over previous
import math

import jax
import jax.numpy as jnp
import numpy as np
from jax.experimental import pallas as pl

_NBINS = 9
_POOL = 7
_GW = 16

_COS = [math.cos(k * math.pi / _NBINS) for k in range(1, _NBINS)]
_SIN = [math.sin(k * math.pi / _NBINS) for k in range(1, _NBINS)]


def _gauss_window(h, w):
    n = jnp.arange(0, _GW, dtype=jnp.float32)
    n = n - n.mean()
    n = n / (_GW // 2)
    wv = jnp.exp(-0.5 * n ** 2)
    g = jnp.outer(wv, wv)
    g = g / g.sum()
    return jnp.tile(g, (h // _GW, w // _GW))


def _hog_body(xp_ref, gk_ref, p_ref, pt_ref, out_ref):
    h = out_ref.shape[2] * _POOL
    w = out_ref.shape[3] * _POOL
    xp = xp_ref[0].astype(jnp.float32)

    sv = xp[0:h, :] + 2.0 * xp[1:h + 1, :] + xp[2:h + 2, :]
    gx = sv[:, 0:w] - sv[:, 2:w + 2]
    sh = xp[:, 0:w] + 2.0 * xp[:, 1:w + 1] + xp[:, 2:w + 2]
    gy = sh[0:h, :] - sh[2:h + 2, :]

    nwg = jnp.sqrt(gx * gx + gy * gy) * gk_ref[...]

    neg = gx < 0.0
    u = jnp.where(neg, -gy, gy)
    v = jnp.abs(gx)
    gx_zero = gx == 0.0

    pooled = []
    sumsq = None
    prev = None
    for k in range(_NBINS):
        if k < _NBINS - 1:
            t_next = ((_COS[k] * v - _SIN[k] * u) >= 0.0) & jnp.logical_not(gx_zero)
            mask = jnp.logical_not(t_next) if prev is None else (prev & jnp.logical_not(t_next))
        else:
            t_next = None
            mask = prev
        masked = jnp.where(mask, nwg, 0.0)
        col = jnp.dot(masked, pt_ref[...], preferred_element_type=jnp.float32,
                      precision=jax.lax.Precision.HIGHEST)
        pb = jnp.dot(p_ref[...], col, preferred_element_type=jnp.float32,
                     precision=jax.lax.Precision.HIGHEST)
        pooled.append(pb)
        sumsq = pb * pb if sumsq is None else sumsq + pb * pb
        prev = t_next

    denom = jnp.maximum(jnp.sqrt(sumsq), 1e-12)
    for k in range(_NBINS):
        out_ref[0, k] = pooled[k] / denom


def kernel(x):
    b, c, h, w = x.shape
    hp, wp = h // _POOL, w // _POOL
    xp = jnp.pad(x, ((0, 0), (0, 0), (1, 1), (1, 1)), mode="reflect")
    xp = xp.reshape(b * c, h + 2, w + 2).astype(jnp.bfloat16)

    gk = _gauss_window(h, w)
    pool_rows = (np.arange(w) // _POOL == np.arange(wp)[:, None]).astype(np.float32)
    p = jnp.asarray(pool_rows)
    pt = jnp.asarray(pool_rows.T)

    out = pl.pallas_call(
        _hog_body,
        grid=(b * c,),
        in_specs=[
            pl.BlockSpec((1, h + 2, w + 2), lambda i: (i, 0, 0)),
            pl.BlockSpec((h, w), lambda i: (0, 0)),
            pl.BlockSpec((hp, h), lambda i: (0, 0)),
            pl.BlockSpec((w, wp), lambda i: (0, 0)),
        ],
        out_specs=pl.BlockSpec((1, _NBINS, hp, wp), lambda i: (i, 0, 0, 0)),
        out_shape=jax.ShapeDtypeStruct((b * c, _NBINS, hp, wp), jnp.float32),
    )(xp, gk, p, pt)
    return out.reshape(b, c, _NBINS, hp, wp)


# speedup vs baseline: 7.1930x; 3.4795x over previous
import math

import jax
import jax.numpy as jnp
import numpy as np
from jax.experimental import pallas as pl

_NBINS = 9
_POOL = 7
_GW = 16

_COS = [math.cos(k * math.pi / _NBINS) for k in range(1, _NBINS)]
_SIN = [math.sin(k * math.pi / _NBINS) for k in range(1, _NBINS)]


def _gauss_window(h, w):
    n = jnp.arange(0, _GW, dtype=jnp.float32)
    n = n - n.mean()
    n = n / (_GW // 2)
    wv = jnp.exp(-0.5 * n ** 2)
    g = jnp.outer(wv, wv)
    g = g / g.sum()
    return jnp.tile(g, (h // _GW, w // _GW))


def _sobel_taps(h, w):
    db = np.zeros((w + 2, 512), np.float32)
    for j in range(w):
        db[j, j] = 1.0
        db[j + 2, j] = -1.0
        db[j, 256 + j] = 1.0
        db[j + 1, 256 + j] = 2.0
        db[j + 2, 256 + j] = 1.0
    return db


def _hog_body(xp_ref, db_ref, gk_ref, p_ref, pt_ref, out_ref):
    h = out_ref.shape[2] * _POOL
    w = out_ref.shape[3] * _POOL
    xp = xp_ref[0]

    g = jnp.dot(xp, db_ref[...], preferred_element_type=jnp.float32)
    hd = g[:, 0:w]
    hs = g[:, 256:256 + w]

    gx = hd[0:h, :] + 2.0 * hd[1:h + 1, :] + hd[2:h + 2, :]
    gy = hs[0:h, :] - hs[2:h + 2, :]

    nwg = jnp.sqrt(gx * gx + gy * gy) * gk_ref[...]

    neg = gx < 0.0
    u = jnp.where(neg, -gy, gy)
    v = jnp.abs(gx)
    gx_zero = gx == 0.0

    pooled = []
    sumsq = None
    prev = None
    for k in range(_NBINS):
        if k < _NBINS - 1:
            t_next = ((_COS[k] * v - _SIN[k] * u) >= 0.0) & jnp.logical_not(gx_zero)
            mask = jnp.logical_not(t_next) if prev is None else (prev & jnp.logical_not(t_next))
        else:
            t_next = None
            mask = prev
        masked = jnp.where(mask, nwg, 0.0)
        rp = jnp.dot(p_ref[...], masked, preferred_element_type=jnp.float32)
        pb = jnp.dot(rp, pt_ref[...], preferred_element_type=jnp.float32)
        pooled.append(pb)
        sumsq = pb * pb if sumsq is None else sumsq + pb * pb
        prev = t_next

    denom = jnp.maximum(jnp.sqrt(sumsq), 1e-12)
    for k in range(_NBINS):
        out_ref[0, k] = pooled[k] / denom


def kernel(x):
    b, c, h, w = x.shape
    hp, wp = h // _POOL, w // _POOL
    xp = jnp.pad(x, ((0, 0), (0, 0), (1, 1), (1, 1)), mode="reflect")
    xp = xp.reshape(b * c, h + 2, w + 2).astype(jnp.bfloat16)

    db = jnp.asarray(_sobel_taps(h, w), jnp.bfloat16)
    gk = _gauss_window(h, w)
    pool_rows = (np.arange(w) // _POOL == np.arange(wp)[:, None]).astype(np.float32)
    p = jnp.asarray(pool_rows)
    pt = jnp.asarray(pool_rows.T)

    out = pl.pallas_call(
        _hog_body,
        grid=(b * c,),
        in_specs=[
            pl.BlockSpec((1, h + 2, w + 2), lambda i: (i, 0, 0)),
            pl.BlockSpec((w + 2, 512), lambda i: (0, 0)),
            pl.BlockSpec((h, w), lambda i: (0, 0)),
            pl.BlockSpec((hp, h), lambda i: (0, 0)),
            pl.BlockSpec((w, wp), lambda i: (0, 0)),
        ],
        out_specs=pl.BlockSpec((1, _NBINS, hp, wp), lambda i: (i, 0, 0, 0)),
        out_shape=jax.ShapeDtypeStruct((b * c, _NBINS, hp, wp), jnp.float32),
    )(xp, db, gk, p, pt)
    return out.reshape(b, c, _NBINS, hp, wp)


# speedup vs baseline: 8.8839x; 1.2351x over previous
import math

import jax
import jax.numpy as jnp
import numpy as np
from jax.experimental import pallas as pl

_NBINS = 9
_POOL = 7
_GW = 16

_NTAN = [-math.tan(k * math.pi / _NBINS) for k in range(1, _NBINS)]


def _gauss_window(h, w):
    n = jnp.arange(0, _GW, dtype=jnp.float32)
    n = n - n.mean()
    n = n / (_GW // 2)
    wv = jnp.exp(-0.5 * n ** 2)
    g = jnp.outer(wv, wv)
    g = g / g.sum()
    return jnp.tile(g, (h // _GW, w // _GW))


def _sobel_taps(h, w):
    db = np.zeros((w + 2, 512), np.float32)
    for j in range(w):
        db[j, j] = 1.0
        db[j + 2, j] = -1.0
        db[j, 256 + j] = 1.0
        db[j + 1, 256 + j] = 2.0
        db[j + 2, 256 + j] = 1.0
    return db


def _hog_body(xp_ref, db_ref, gk_ref, p_ref, pt_ref, out_ref):
    h = out_ref.shape[2] * _POOL
    w = out_ref.shape[3] * _POOL
    xp = xp_ref[0]

    g = jnp.dot(xp, db_ref[...], preferred_element_type=jnp.float32)
    hd = g[:, 0:w]
    hs = g[:, 256:256 + w]

    gx = hd[0:h, :] + 2.0 * hd[1:h + 1, :] + hd[2:h + 2, :]
    gy = hs[0:h, :] - hs[2:h + 2, :]

    nwg = jnp.sqrt(gx * gx + gy * gy) * gk_ref[...]

    neg = gx < 0.0
    u = jnp.where(neg, -gy, gy)
    v = jnp.abs(gx)
    u = jnp.where(gx == 0.0, jnp.abs(gy), u)

    pooled = []
    sumsq = None
    prev = None
    for k in range(_NBINS):
        if k < _NBINS - 1:
            e = _NTAN[k] * u + v
            t_next = (e >= 0.0) if k < 4 else (e <= 0.0)
            mask = jnp.logical_not(t_next) if prev is None else (prev & jnp.logical_not(t_next))
        else:
            t_next = None
            mask = prev
        masked = jnp.where(mask, nwg, 0.0)
        rp = jnp.dot(p_ref[...], masked, preferred_element_type=jnp.float32)
        pb = jnp.dot(rp, pt_ref[...], preferred_element_type=jnp.float32)
        pooled.append(pb)
        sumsq = pb * pb if sumsq is None else sumsq + pb * pb
        prev = t_next

    denom = jnp.maximum(jnp.sqrt(sumsq), 1e-12)
    for k in range(_NBINS):
        out_ref[0, k] = pooled[k] / denom


def kernel(x):
    b, c, h, w = x.shape
    hp, wp = h // _POOL, w // _POOL
    xp = jnp.pad(x, ((0, 0), (0, 0), (1, 1), (1, 1)), mode="reflect")
    xp = xp.reshape(b * c, h + 2, w + 2).astype(jnp.bfloat16)

    db = jnp.asarray(_sobel_taps(h, w), jnp.bfloat16)
    gk = _gauss_window(h, w)
    pool_rows = (np.arange(w) // _POOL == np.arange(wp)[:, None]).astype(np.float32)
    p = jnp.asarray(pool_rows)
    pt = jnp.asarray(pool_rows.T)

    out = pl.pallas_call(
        _hog_body,
        grid=(b * c,),
        in_specs=[
            pl.BlockSpec((1, h + 2, w + 2), lambda i: (i, 0, 0)),
            pl.BlockSpec((w + 2, 512), lambda i: (0, 0)),
            pl.BlockSpec((h, w), lambda i: (0, 0)),
            pl.BlockSpec((hp, h), lambda i: (0, 0)),
            pl.BlockSpec((w, wp), lambda i: (0, 0)),
        ],
        out_specs=pl.BlockSpec((1, _NBINS, hp, wp), lambda i: (i, 0, 0, 0)),
        out_shape=jax.ShapeDtypeStruct((b * c, _NBINS, hp, wp), jnp.float32),
    )(xp, db, gk, p, pt)
    return out.reshape(b, c, _NBINS, hp, wp)


# speedup vs baseline: 9.8938x; 1.1137x over previous
import math

import jax
import jax.numpy as jnp
import numpy as np
from jax.experimental import pallas as pl

_NBINS = 9
_POOL = 7
_GW = 16

_NTAN = [-math.tan(k * math.pi / _NBINS) for k in range(1, _NBINS)]


def _gauss_window(h, w):
    n = jnp.arange(0, _GW, dtype=jnp.float32)
    n = n - n.mean()
    n = n / (_GW // 2)
    wv = jnp.exp(-0.5 * n ** 2)
    g = jnp.outer(wv, wv)
    g = g / g.sum()
    return jnp.tile(g, (h // _GW, w // _GW))


def _sobel_taps(h, w):
    db = np.zeros((w + 2, 512), np.float32)
    for j in range(w):
        db[j, j] = 1.0
        db[j + 2, j] = -1.0
        db[j, 256 + j] = 1.0
        db[j + 1, 256 + j] = 2.0
        db[j + 2, 256 + j] = 1.0
    return db


_STRIP = 224


def _hog_body(xp_ref, db_ref, gk_ref, p_ref, pt_ref, out_ref):
    h = out_ref.shape[2] * _POOL
    w = out_ref.shape[3] * _POOL
    for im in range(out_ref.shape[0]):
        xp = xp_ref[im]

        g = jnp.dot(xp, db_ref[...], preferred_element_type=jnp.float32)
        hd = g[:, 0:w]
        hs = g[:, 256:256 + w]

        gx = hd[0:h, :] + 2.0 * hd[1:h + 1, :] + hd[2:h + 2, :]
        gy = hs[0:h, :] - hs[2:h + 2, :]

        nwg = jnp.sqrt(gx * gx + gy * gy) * gk_ref[...]

        neg = gx < 0.0
        u = jnp.where(neg, -gy, gy)
        v = jnp.abs(gx)
        u = jnp.where(gx == 0.0, jnp.abs(gy), u)

        pooled = []
        sumsq = None
        prev = None
        for k in range(_NBINS):
            if k < _NBINS - 1:
                e = _NTAN[k] * u + v
                t_next = (e >= 0.0) if k < 4 else (e <= 0.0)
                mask = jnp.logical_not(t_next) if prev is None else (prev & jnp.logical_not(t_next))
            else:
                t_next = None
                mask = prev
            masked = jnp.where(mask, nwg, 0.0)
            rp = jnp.dot(p_ref[...], masked, preferred_element_type=jnp.float32)
            pb = jnp.dot(rp, pt_ref[...], preferred_element_type=jnp.float32)
            pooled.append(pb)
            sumsq = pb * pb if sumsq is None else sumsq + pb * pb
            prev = t_next

        inv = 1.0 / jnp.maximum(jnp.sqrt(sumsq), 1e-12)
        for k in range(_NBINS):
            out_ref[im, k] = pooled[k] * inv


def kernel(x):
    b, c, h, w = x.shape
    hp, wp = h // _POOL, w // _POOL
    xp = jnp.pad(x, ((0, 0), (0, 0), (1, 1), (1, 1)), mode="reflect")
    xp = xp.reshape(b * c, h + 2, w + 2).astype(jnp.bfloat16)

    db = jnp.asarray(_sobel_taps(h, w), jnp.bfloat16)
    gk = _gauss_window(h, w)
    pool_rows = (np.arange(w) // _POOL == np.arange(wp)[:, None]).astype(np.float32)
    p = jnp.asarray(pool_rows)
    pt = jnp.asarray(pool_rows.T)

    npp = 4
    out = pl.pallas_call(
        _hog_body,
        grid=(b * c // npp,),
        in_specs=[
            pl.BlockSpec((npp, h + 2, w + 2), lambda i: (i, 0, 0)),
            pl.BlockSpec((w + 2, 512), lambda i: (0, 0)),
            pl.BlockSpec((h, w), lambda i: (0, 0)),
            pl.BlockSpec((wp, w), lambda i: (0, 0)),
            pl.BlockSpec((w, wp), lambda i: (0, 0)),
        ],
        out_specs=pl.BlockSpec((npp, _NBINS, hp, wp), lambda i: (i, 0, 0, 0)),
        out_shape=jax.ShapeDtypeStruct((b * c, _NBINS, hp, wp), jnp.float32),
    )(xp, db, gk, p, pt)
    return out.reshape(b, c, _NBINS, hp, wp)


# speedup vs baseline: 10.0780x; 1.0186x over previous
import math

import jax
import jax.numpy as jnp
import numpy as np
from jax.experimental import pallas as pl

_NBINS = 9
_POOL = 7
_GW = 16

_NTAN = [-math.tan(k * math.pi / _NBINS) for k in range(1, _NBINS)]


def _gauss_window(h, w):
    n = jnp.arange(0, _GW, dtype=jnp.float32)
    n = n - n.mean()
    n = n / (_GW // 2)
    wv = jnp.exp(-0.5 * n ** 2)
    g = jnp.outer(wv, wv)
    g = g / g.sum()
    return jnp.tile(g, (h // _GW, w // _GW))


def _sobel_taps(h, w):
    db = np.zeros((w + 2, 512), np.float32)
    for j in range(w):
        db[j, j] = 1.0
        db[j + 2, j] = -1.0
        db[j, 256 + j] = 1.0
        db[j + 1, 256 + j] = 2.0
        db[j + 2, 256 + j] = 1.0
    return db


_STRIP = 224


def _hog_body(xp_ref, db_ref, gk_ref, p_ref, pt_ref, out_ref):
    h = out_ref.shape[2] * _POOL
    w = out_ref.shape[3] * _POOL
    for im in range(out_ref.shape[0]):
        xp = xp_ref[im]

        g = jnp.dot(xp, db_ref[...], preferred_element_type=jnp.float32)
        hd = g[:, 0:w]
        hs = g[:, 256:256 + w]

        gx = hd[0:h, :] + 2.0 * hd[1:h + 1, :] + hd[2:h + 2, :]
        gy = hs[0:h, :] - hs[2:h + 2, :]

        nwg = jnp.sqrt(gx * gx + gy * gy) * gk_ref[...]

        neg = gx < 0.0
        u = jnp.where(neg, -gy, gy)
        v = jnp.abs(gx)
        u = jnp.where(gx == 0.0, jnp.abs(gy), u)

        pooled = []
        sumsq = None
        prev = None
        for k in range(_NBINS):
            if k < _NBINS - 1:
                e = _NTAN[k] * u + v
                t_next = (e >= 0.0) if k < 4 else (e <= 0.0)
                mask = jnp.logical_not(t_next) if prev is None else (prev & jnp.logical_not(t_next))
            else:
                t_next = None
                mask = prev
            masked = jnp.where(mask, nwg, 0.0)
            rp = jnp.dot(p_ref[...], masked, preferred_element_type=jnp.float32)
            pb = jnp.dot(rp, pt_ref[...], preferred_element_type=jnp.float32)
            pooled.append(pb)
            sumsq = pb * pb if sumsq is None else sumsq + pb * pb
            prev = t_next

        inv = 1.0 / jnp.maximum(jnp.sqrt(sumsq), 1e-12)
        for k in range(_NBINS):
            out_ref[im, k] = pooled[k] * inv


def kernel(x):
    b, c, h, w = x.shape
    hp, wp = h // _POOL, w // _POOL
    xp = jnp.pad(x, ((0, 0), (0, 0), (1, 1), (1, 1)), mode="reflect")
    xp = xp.reshape(b * c, h + 2, w + 2).astype(jnp.bfloat16)

    db = jnp.asarray(_sobel_taps(h, w), jnp.bfloat16)
    gk = _gauss_window(h, w)
    pool_rows = (np.arange(w) // _POOL == np.arange(wp)[:, None]).astype(np.float32)
    p = jnp.asarray(pool_rows)
    pt = jnp.asarray(pool_rows.T)

    npp = 8
    out = pl.pallas_call(
        _hog_body,
        grid=(b * c // npp,),
        in_specs=[
            pl.BlockSpec((npp, h + 2, w + 2), lambda i: (i, 0, 0)),
            pl.BlockSpec((w + 2, 512), lambda i: (0, 0)),
            pl.BlockSpec((h, w), lambda i: (0, 0)),
            pl.BlockSpec((wp, w), lambda i: (0, 0)),
            pl.BlockSpec((w, wp), lambda i: (0, 0)),
        ],
        out_specs=pl.BlockSpec((npp, _NBINS, hp, wp), lambda i: (i, 0, 0, 0)),
        out_shape=jax.ShapeDtypeStruct((b * c, _NBINS, hp, wp), jnp.float32),
    )(xp, db, gk, p, pt)
    return out.reshape(b, c, _NBINS, hp, wp)


# speedup vs baseline: 10.1063x; 1.0028x over previous
import math

import jax
import jax.numpy as jnp
import numpy as np
from jax.experimental import pallas as pl

_NBINS = 9
_POOL = 7
_GW = 16

_NTAN = [-math.tan(k * math.pi / _NBINS) for k in range(1, _NBINS)]


def _gauss_window(h, w):
    n = jnp.arange(0, _GW, dtype=jnp.float32)
    n = n - n.mean()
    n = n / (_GW // 2)
    wv = jnp.exp(-0.5 * n ** 2)
    g = jnp.outer(wv, wv)
    g = g / g.sum()
    return jnp.tile(g, (h // _GW, w // _GW))


def _sobel_taps(h, w):
    db = np.zeros((w + 2, 512), np.float32)
    for j in range(w):
        db[j, j] = 1.0
        db[j + 2, j] = -1.0
        db[j, 256 + j] = 1.0
        db[j + 1, 256 + j] = 2.0
        db[j + 2, 256 + j] = 1.0
    return db


def _hog_body(xp_ref, db_ref, gk_ref, p_ref, pt_ref, out_ref):
    h = out_ref.shape[2] * _POOL
    w = out_ref.shape[3] * _POOL
    for im in range(out_ref.shape[0]):
        xp = xp_ref[im]

        g = jnp.dot(xp, db_ref[...], preferred_element_type=jnp.float32)
        hd = g[:, 0:w]
        hs = g[:, 256:256 + w]

        gx = hd[0:h, :] + 2.0 * hd[1:h + 1, :] + hd[2:h + 2, :]
        gy = hs[0:h, :] - hs[2:h + 2, :]

        nwg = jnp.sqrt(gx * gx + gy * gy) * gk_ref[...]

        neg = gx < 0.0
        u = jnp.where(neg, -gy, gy)
        v = jnp.abs(gx)
        u = jnp.where(gx == 0.0, jnp.abs(gy), u)

        pooled = []
        sumsq = None
        prev = None
        for k in range(_NBINS):
            if k < _NBINS - 1:
                e = _NTAN[k] * u + v
                t_next = (e >= 0.0) if k < 4 else (e <= 0.0)
                mask = jnp.logical_not(t_next) if prev is None else (prev & jnp.logical_not(t_next))
            else:
                t_next = None
                mask = prev
            masked = jnp.where(mask, nwg, 0.0)
            rp = jnp.dot(p_ref[...], masked, preferred_element_type=jnp.float32)
            pb = jnp.dot(rp, pt_ref[...], preferred_element_type=jnp.float32)
            pooled.append(pb)
            sumsq = pb * pb if sumsq is None else sumsq + pb * pb
            prev = t_next

        inv = 1.0 / jnp.maximum(jnp.sqrt(sumsq), 1e-12)
        for k in range(_NBINS):
            out_ref[im, k] = pooled[k] * inv


def kernel(x):
    b, c, h, w = x.shape
    hp, wp = h // _POOL, w // _POOL
    xp = jnp.pad(x, ((0, 0), (0, 0), (1, 1), (1, 1)), mode="reflect")
    xp = xp.reshape(b * c, h + 2, w + 2).astype(jnp.bfloat16)

    db = jnp.asarray(_sobel_taps(h, w), jnp.bfloat16)
    gk = _gauss_window(h, w)
    pool_rows = (np.arange(w) // _POOL == np.arange(wp)[:, None]).astype(np.float32)
    p = jnp.asarray(pool_rows)
    pt = jnp.asarray(pool_rows.T)

    npp = next(n for n in (12, 8, 6, 4, 3, 2, 1) if (b * c) % n == 0)
    out = pl.pallas_call(
        _hog_body,
        grid=(b * c // npp,),
        in_specs=[
            pl.BlockSpec((npp, h + 2, w + 2), lambda i: (i, 0, 0)),
            pl.BlockSpec((w + 2, 512), lambda i: (0, 0)),
            pl.BlockSpec((h, w), lambda i: (0, 0)),
            pl.BlockSpec((wp, w), lambda i: (0, 0)),
            pl.BlockSpec((w, wp), lambda i: (0, 0)),
        ],
        out_specs=pl.BlockSpec((npp, _NBINS, hp, wp), lambda i: (i, 0, 0, 0)),
        out_shape=jax.ShapeDtypeStruct((b * c, _NBINS, hp, wp), jnp.float32),
    )(xp, db, gk, p, pt)
    return out.reshape(b, c, _NBINS, hp, wp)
